# baseline (speedup 1.0000x reference)
import functools

import jax
import jax.numpy as jnp
from jax import lax
from jax.experimental import pallas as pl
from jax.experimental.pallas import tpu as pltpu
from jax.experimental.pallas import tpu_sc as plsc

D = 128
L = 32

_NC = 2
_NS = 16
_NW = _NC * _NS

_CH = 256
_CB = _CH // L
_OB = 64


def _proj_kernel(feat_ref, a_ref, w1a_ref, p_ref, f1_ref):
    f = feat_ref[...]
    p_ref[...] = jnp.dot(f, a_ref[...], preferred_element_type=jnp.float32)
    f1_ref[...] = jnp.dot(f, w1a_ref[...], preferred_element_type=jnp.float32)


def _ctab_kernel(r_ref, bm_ref, bgv_ref, c_ref):
    c_ref[...] = (
        jnp.dot(r_ref[...], bm_ref[...], preferred_element_type=jnp.float32)
        + bgv_ref[...]
    )


def _final_kernel(s1_ref, n_ref, w1b_ref, b1_ref, out_ref):
    comb = (s1_ref[...]
            + jnp.dot(n_ref[...], w1b_ref[...], preferred_element_type=jnp.float32)
            + b1_ref[...])
    out_ref[...] = jnp.maximum(comb, 0.0)


def _fire_chunk(p_hbm, uvidx_v, buf, sem, t):
    descs = []
    for j in range(_CH // 128):
        descs.append(
            pltpu.async_copy(p_hbm.at[uvidx_v.at[t * (_CH // 128) + j]],
                             buf.at[pl.ds(j * 128, 128)], sem))
    return descs


def _consume_chunk(rvals_v, c_v, buf, obuf, t):
    def b_body(b, carry):
        row0 = b * L
        rbase = t * _CH + row0
        acc = [jnp.zeros((16,), jnp.float32) for _ in range(D // 16)]
        for g in range(L // 16):
            r16 = rvals_v[pl.ds(rbase + g * 16, 16)]
            for li in range(16):
                l = g * 16 + li
                r_i = r16[li]
                crow = c_v.at[r_i]
                prow = buf.at[row0 + l]
                for j in range(D // 16):
                    p = prow[pl.ds(j * 16, 16)]
                    c = crow[pl.ds(j * 16, 16)]
                    acc[j] = acc[j] + jnp.maximum(p + c, 0.0)
        ob = (t % (_OB // _CB)) * _CB + b
        orow = obuf.at[ob]
        for j in range(D // 16):
            orow[pl.ds(j * 16, 16)] = acc[j] * (1.0 / L)
        return carry
    return b_body


def _sc_fused_body(p_hbm, f1_hbm, uv_hbm, r_hbm, c_hbm, nodes_hbm,
                   neigh_out, s_out,
                   uvidx_v, rvals_v, c_v, buf_a, buf_b, obuf_a,
                   nidx_v, gsem):
    wid = lax.axis_index("s") * _NC + lax.axis_index("c")
    bpw = uv_hbm.shape[1]
    nch = (bpw * 128) // _CH
    spw = nodes_hbm.shape[0] * 128 // _NW

    pltpu.sync_copy(uv_hbm.at[wid], uvidx_v)
    pltpu.sync_copy(r_hbm.at[pl.ds(wid * bpw * 128, bpw * 128)], rvals_v)
    pltpu.sync_copy(c_hbm, c_v)

    for d in _fire_chunk(p_hbm, uvidx_v, buf_a, gsem, 0):
        d.wait()

    opc = _OB // _CB

    def two_chunks(i, carry):
        t0 = i * 2
        d1 = _fire_chunk(p_hbm, uvidx_v, buf_b, gsem, t0 + 1)
        lax.fori_loop(0, _CB,
                      _consume_chunk(rvals_v, c_v, buf_a, obuf_a, t0), 0)
        for d in d1:
            d.wait()

        @pl.when(i < nch // 2 - 1)
        def _():
            _fire_chunk(p_hbm, uvidx_v, buf_a, gsem, t0 + 2)

        lax.fori_loop(0, _CB,
                      _consume_chunk(rvals_v, c_v, buf_b, obuf_a, t0 + 1), 0)

        @pl.when(i < nch // 2 - 1)
        def _():
            pltpu.make_async_copy(
                p_hbm.at[pl.ds(0, _CH)], buf_a, gsem).wait()

        @pl.when((t0 + 2) % opc == 0)
        def _():
            ob_base = wid * (bpw * 128 // L) + (t0 + 2 - opc) * _CB
            pltpu.sync_copy(obuf_a, neigh_out.at[pl.ds(ob_base, _OB)])
        return carry

    lax.fori_loop(0, nch // 2, two_chunks, 0)

    pltpu.sync_copy(nodes_hbm.at[pl.ds(wid * (spw // 128), spw // 128)],
                    nidx_v)
    for h in range(spw // _CH):
        descs = [
            pltpu.async_copy(
                f1_hbm.at[nidx_v.at[h * (_CH // 128) + j]],
                buf_a.at[pl.ds(j * 128, 128)], gsem)
            for j in range(_CH // 128)
        ]
        for d in descs:
            d.wait()
        pltpu.sync_copy(buf_a, s_out.at[pl.ds(wid * spw + h * _CH, _CH)])


def kernel(nodes, history_uv, history_r, feat_table, r_table, W_gv, b_gv, W1, b1):
    B = nodes.shape[0]
    V = feat_table.shape[0]
    BL = B * L
    bpw = BL // _NW // 128

    nodes_i = nodes.astype(jnp.int32).reshape(B // 128, 128)
    uv_i = history_uv.astype(jnp.int32).reshape(_NW, bpw, 128)
    r_flat = history_r.astype(jnp.int32).reshape(BL)

    A = W_gv[:D]
    Bm = W_gv[D:]
    W1a = W1[:D]
    W1b = W1[D:]
    r_pad = jnp.pad(r_table, ((0, 8 - r_table.shape[0]), (0, 0)))

    rb = 10000
    P, F1 = pl.pallas_call(
        _proj_kernel,
        grid=(V // rb,),
        in_specs=[
            pl.BlockSpec((rb, D), lambda i: (i, 0)),
            pl.BlockSpec((D, D), lambda i: (0, 0)),
            pl.BlockSpec((D, D), lambda i: (0, 0)),
        ],
        out_specs=[
            pl.BlockSpec((rb, D), lambda i: (i, 0)),
            pl.BlockSpec((rb, D), lambda i: (i, 0)),
        ],
        out_shape=[jax.ShapeDtypeStruct((V, D), jnp.float32)] * 2,
    )(feat_table, A, W1a)

    C = pl.pallas_call(
        _ctab_kernel,
        out_shape=jax.ShapeDtypeStruct((8, D), jnp.float32),
    )(r_pad, Bm, b_gv.reshape(1, D))

    mesh = plsc.VectorSubcoreMesh(core_axis_name="c", subcore_axis_name="s")
    sc_fused = functools.partial(
        pl.kernel,
        mesh=mesh,
        out_type=(
            jax.ShapeDtypeStruct((B, D), jnp.float32),
            jax.ShapeDtypeStruct((B, D), jnp.float32),
        ),
        scratch_types=[
            pltpu.VMEM((bpw, 128), jnp.int32),
            pltpu.VMEM((bpw * 128,), jnp.int32),
            pltpu.VMEM((8, D), jnp.float32),
            pltpu.VMEM((_CH, D), jnp.float32),
            pltpu.VMEM((_CH, D), jnp.float32),
            pltpu.VMEM((_OB, D), jnp.float32),
            pltpu.VMEM((4, 128), jnp.int32),
            pltpu.SemaphoreType.DMA,
        ],
    )(_sc_fused_body)
    neigh, S1 = sc_fused(P, F1, uv_i, r_flat, C, nodes_i)

    out = pl.pallas_call(
        _final_kernel,
        grid=(B // 2048,),
        in_specs=[
            pl.BlockSpec((2048, D), lambda i: (i, 0)),
            pl.BlockSpec((2048, D), lambda i: (i, 0)),
            pl.BlockSpec((D, D), lambda i: (0, 0)),
            pl.BlockSpec((1, D), lambda i: (0, 0)),
        ],
        out_specs=pl.BlockSpec((2048, D), lambda i: (i, 0)),
        out_shape=jax.ShapeDtypeStruct((B, D), jnp.float32),
    )(S1, neigh, W1b, b1.reshape(1, D))
    return out
